# baseline (speedup 1.0000x reference)
import functools

import jax
import jax.numpy as jnp
from jax import lax
from jax.experimental import pallas as pl
from jax.experimental.pallas import tpu as pltpu

VOCAB = 64
DIM = 64
TOT = 4096 * 200
BLK2 = 1024
GRID = TOT // (2 * BLK2)


def _tc_body(idx_ref, table2_ref, out_ref):
    ids = idx_ref[...]
    even = lax.slice(ids, (0, 0), (BLK2, 1))
    odd = lax.slice(ids, (0, 1), (BLK2, 2))
    iota = lax.broadcasted_iota(jnp.int32, (1, 2 * VOCAB), 1)
    oh = ((even == iota) | (odd == iota - VOCAB)).astype(jnp.float32)
    out_ref[...] = jnp.dot(
        oh, table2_ref[...], preferred_element_type=jnp.float32
    )


@jax.jit
def _tc_lookup(flat_idx, table):
    table2 = jnp.zeros((2 * VOCAB, 2 * DIM), jnp.float32)
    table2 = table2.at[:VOCAB, :DIM].set(table)
    table2 = table2.at[VOCAB:, DIM:].set(table)
    return pl.pallas_call(
        _tc_body,
        grid=(GRID,),
        in_specs=[
            pl.BlockSpec((BLK2, 2), lambda i: (i, 0)),
            pl.BlockSpec((2 * VOCAB, 2 * DIM), lambda i: (0, 0)),
        ],
        out_specs=pl.BlockSpec((BLK2, 2 * DIM), lambda i: (i, 0)),
        out_shape=jax.ShapeDtypeStruct((TOT // 2, 2 * DIM), jnp.float32),
    )(flat_idx.reshape(TOT // 2, 2), table2)


def kernel(indices, table):
    out = _tc_lookup(indices.reshape(TOT), table)
    return out.reshape(indices.shape + (DIM,))
